# Initial kernel scaffold; baseline (speedup 1.0000x reference)
import jax
import jax.numpy as jnp
from jax.experimental import pallas as pl


def kernel(keys_pop, Q):
    raise NotImplementedError("write your pallas kernel here")


# speedup vs baseline: 1.3104x; 1.3104x over previous
import functools

import jax
import jax.numpy as jnp
from jax.experimental import pallas as pl

POP_ = 128
GENE_ = 4096
BK_ = 512


def _quadform_kernel(x_ref, q_ref, out_ref):
    k = pl.program_id(0)
    x = x_ref[...]
    q = q_ref[...]
    y = jnp.dot(
        x.astype(jnp.bfloat16),
        q.astype(jnp.bfloat16),
        preferred_element_type=jnp.float32,
    )
    xk = x_ref[:, pl.ds(k * BK_, BK_)]
    partial = jnp.sum(y * xk, axis=1)

    @pl.when(k == 0)
    def _init():
        out_ref[...] = partial[None, :]

    @pl.when(k > 0)
    def _acc():
        out_ref[...] += partial[None, :]


@jax.jit
def kernel(keys_pop, Q):
    out = pl.pallas_call(
        _quadform_kernel,
        grid=(GENE_ // BK_,),
        in_specs=[
            pl.BlockSpec((POP_, GENE_), lambda k: (0, 0)),
            pl.BlockSpec((GENE_, BK_), lambda k: (0, k)),
        ],
        out_specs=pl.BlockSpec((1, POP_), lambda k: (0, 0)),
        out_shape=jax.ShapeDtypeStruct((1, POP_), jnp.float32),
    )(keys_pop, Q)
    return out[0]
